# baseline (speedup 1.0000x reference)
import functools

import jax
import jax.numpy as jnp
from jax import lax
from jax.experimental import pallas as pl
from jax.experimental.pallas import tpu as pltpu
from jax.experimental.pallas import tpu_sc as plsc

D = 2048
H = 256
E = 8
B = 2
S = 2048

TM = 512
TC = 128
PH1 = B * S // TM
P = S // TM


def _gate_kernel(x_ref, w_ref, gb_ref, w1_ref, b1_ref, w2_ref, b2_ref,
                 gated_ref, logits_ref, wb_s, psum_s):
    i = pl.program_id(0)

    @pl.when(i == 0)
    def _cast_w():
        wb_s[...] = w_ref[...].astype(jnp.bfloat16)

    xt = x_ref[...]
    z = jnp.dot(xt.astype(jnp.bfloat16), wb_s[...],
                preferred_element_type=jnp.float32) + gb_ref[...]
    g = xt * jax.nn.sigmoid(z)
    gated_ref[...] = g.astype(jnp.bfloat16)
    colsum = jnp.sum(g, axis=0, keepdims=True)

    for r in range(PH1):
        if r in (P - 1, 2 * P - 1):
            continue
        @pl.when(i == r)
        def _store(r=r):
            psum_s[r:r + 1, :] = colsum

    for bb, last in ((0, P - 1), (1, 2 * P - 1)):
        @pl.when(i == last)
        def _logits(bb=bb, last=last):
            ps = psum_s[...]
            prev = jnp.sum(ps[last - (P - 1):last, :], axis=0, keepdims=True)
            pooled = (prev + colsum) * (1.0 / S)
            h = jnp.tanh(
                jnp.dot(pooled.astype(jnp.bfloat16), w1_ref[...],
                        preferred_element_type=jnp.float32) + b1_ref[...])
            logits_ref[bb:bb + 1, :] = (
                jnp.dot(h.astype(jnp.bfloat16),
                        w2_ref[...].astype(jnp.bfloat16),
                        preferred_element_type=jnp.float32) + b2_ref[...])


def _sc_router(logits_hbm, gains_hbm, probs_hbm, lg_v, gn_v, pr_v):
    cid = lax.axis_index("c")
    sid = lax.axis_index("s")

    @pl.when(jnp.logical_and(cid == 0, sid == 0))
    def _():
        pltpu.sync_copy(logits_hbm, lg_v)
        lg = lg_v[...]
        iot = lax.iota(jnp.int32, 16)
        mode = "promise_in_bounds"

        def _bfly(v, op):
            for sh in (1, 2, 4):
                v = op(v, v.at[jnp.bitwise_xor(iot, sh)].get(mode=mode))
            return v

        v1 = _bfly(lg, jnp.maximum)
        i1 = _bfly(jnp.where(lg == v1, iot, 99), jnp.minimum)
        m1 = iot == i1
        lgm = jnp.where(m1, jnp.float32(-jnp.inf), lg)
        v2 = _bfly(lgm, jnp.maximum)
        i2 = _bfly(jnp.where(lgm == v2, iot, 99), jnp.minimum)
        m2 = iot == i2
        e2 = jnp.exp(v2 - v1)
        w1 = 1.0 / (1.0 + e2)
        w2 = e2 * w1
        gn_v[...] = jnp.where(m1, w1, 0.0) + jnp.where(m2, w2, 0.0)
        ex = jnp.exp(lg - v1)
        pr_v[...] = ex / _bfly(ex, jnp.add)
        pltpu.sync_copy(gn_v, gains_hbm)
        pltpu.sync_copy(pr_v, probs_hbm)


def _bcast_kernel(gains_ref, gated_ref, out_ref):
    g = gated_ref[0].astype(jnp.float32)
    gv = gains_ref[0]
    for e in range(E):
        out_ref[e, 0] = g * gv[0, e]


def kernel(x, gate_W, gate_b, W1, b1, W2, b2):
    xf = x.reshape(B * S, D)

    gated, logits = pl.pallas_call(
        _gate_kernel,
        grid=(PH1,),
        in_specs=[
            pl.BlockSpec((TM, D), lambda i: (i, 0)),
            pl.BlockSpec((D, D), lambda i: (0, 0)),
            pl.BlockSpec((1, D), lambda i: (0, 0)),
            pl.BlockSpec((D, H), lambda i: (0, 0)),
            pl.BlockSpec((1, H), lambda i: (0, 0)),
            pl.BlockSpec((H, E), lambda i: (0, 0)),
            pl.BlockSpec((1, E), lambda i: (0, 0)),
        ],
        out_specs=[
            pl.BlockSpec((TM, D), lambda i: (i, 0)),
            pl.BlockSpec((B, E), lambda i: (0, 0)),
        ],
        out_shape=[
            jax.ShapeDtypeStruct((B * S, D), jnp.bfloat16),
            jax.ShapeDtypeStruct((B, E), jnp.float32),
        ],
        scratch_shapes=[
            pltpu.VMEM((D, D), jnp.bfloat16),
            pltpu.VMEM((PH1, D), jnp.float32),
        ],
        compiler_params=pltpu.CompilerParams(
            dimension_semantics=("arbitrary",)),
    )(xf, gate_W, gate_b.reshape(1, D), W1.astype(jnp.bfloat16),
      b1.reshape(1, H), W2, b2.reshape(1, E))

    sc_router = functools.partial(
        pl.kernel,
        mesh=plsc.VectorSubcoreMesh(core_axis_name="c", subcore_axis_name="s"),
        out_type=[
            jax.ShapeDtypeStruct((B * E,), jnp.float32),
            jax.ShapeDtypeStruct((B * E,), jnp.float32),
        ],
        scratch_types=[
            pltpu.VMEM((B * E,), jnp.float32),
            pltpu.VMEM((B * E,), jnp.float32),
            pltpu.VMEM((B * E,), jnp.float32),
        ],
    )(_sc_router)
    gains16, probs16 = sc_router(logits.reshape(B * E))

    routed = pl.pallas_call(
        _bcast_kernel,
        grid=(B, S // TC),
        in_specs=[
            pl.BlockSpec((1, 1, E), lambda b, i: (b, 0, 0)),
            pl.BlockSpec((1, TC, D), lambda b, i: (b, i, 0)),
        ],
        out_specs=pl.BlockSpec((E, 1, TC, D), lambda b, i: (0, b, i, 0)),
        out_shape=jax.ShapeDtypeStruct((E, B, S, D), jnp.float32),
        compiler_params=pltpu.CompilerParams(
            dimension_semantics=("parallel", "parallel")),
    )(gains16.reshape(B, 1, E), gated.reshape(B, S, D))

    return routed, probs16.reshape(B, E)


# speedup vs baseline: 1.2119x; 1.2119x over previous
import jax
import jax.numpy as jnp
from jax.experimental import pallas as pl
from jax.experimental.pallas import tpu as pltpu

D = 2048
H = 256
E = 8
B = 2
S = 2048

TM = 512
TC = 128
PH1 = B * S // TM
P = S // TM
NSB = S // TC
NOUT = B * NSB


def _mega_kernel(x_ref, w_ref, gb_ref, w1_ref, b1_ref, w2_ref, b2_ref,
                 routed_ref, probs_ref,
                 gated_s, psum_s, gains_s):
    i = pl.program_id(0)

    @pl.when(i < PH1)
    def _mm():
        xt = x_ref[...]
        z = jnp.dot(xt.astype(jnp.bfloat16), w_ref[...],
                    preferred_element_type=jnp.float32) + gb_ref[...]
        g = xt * jax.nn.sigmoid(z)
        gated_s[pl.ds(i * TM, TM), :] = g.astype(jnp.bfloat16)
        colsum = jnp.sum(g, axis=0, keepdims=True)

        for r in range(PH1):
            if r in (P - 1, 2 * P - 1):
                continue
            @pl.when(i == r)
            def _store(r=r):
                psum_s[r:r + 1, :] = colsum

        for bb, last in ((0, P - 1), (1, 2 * P - 1)):
            @pl.when(i == last)
            def _router(bb=bb, last=last):
                ps = psum_s[...]
                prev = jnp.sum(ps[last - (P - 1):last, :], axis=0,
                               keepdims=True)
                pooled = (prev + colsum) * (1.0 / S)
                h = jnp.tanh(
                    jnp.dot(pooled.astype(jnp.bfloat16), w1_ref[...],
                            preferred_element_type=jnp.float32) + b1_ref[...])
                logits = (jnp.dot(h.astype(jnp.bfloat16),
                                  w2_ref[...].astype(jnp.bfloat16),
                                  preferred_element_type=jnp.float32)
                          + b2_ref[...])
                ids = jax.lax.broadcasted_iota(jnp.int32, (1, E), 1)
                v1 = jnp.max(logits, axis=1, keepdims=True)
                i1 = jnp.min(jnp.where(logits == v1, ids, E),
                             axis=1, keepdims=True)
                m1 = ids == i1
                masked = jnp.where(m1, -jnp.inf, logits)
                v2 = jnp.max(masked, axis=1, keepdims=True)
                i2 = jnp.min(jnp.where(masked == v2, ids, E),
                             axis=1, keepdims=True)
                m2 = ids == i2
                ex = jnp.exp(logits - v1)
                probs_ref[bb:bb + 1, :] = ex / jnp.sum(ex, axis=1,
                                                       keepdims=True)
                e2 = jnp.exp(v2 - v1)
                wt1 = 1.0 / (1.0 + e2)
                wt2 = e2 * wt1
                gains_s[bb:bb + 1, :] = (jnp.where(m1, wt1, 0.0)
                                         + jnp.where(m2, wt2, 0.0))

    @pl.when(i >= P)
    def _out():
        jj = i - P
        bb = jj // NSB
        g = gated_s[pl.ds(jj * TC, TC), :].astype(jnp.float32)
        gv = gains_s[...]
        rid = jax.lax.broadcasted_iota(jnp.int32, (B, E), 0)
        row = jnp.sum(jnp.where(rid == bb, gv, 0.0), axis=0,
                      keepdims=True)
        for e in range(E):
            routed_ref[e, 0] = g * row[0, e]


def kernel(x, gate_W, gate_b, W1, b1, W2, b2):
    xf = x.reshape(B * S, D)

    routed, probs = pl.pallas_call(
        _mega_kernel,
        grid=(P + NOUT,),
        in_specs=[
            pl.BlockSpec((TM, D), lambda i: (jnp.minimum(i, PH1 - 1), 0)),
            pl.BlockSpec((D, D), lambda i: (0, 0)),
            pl.BlockSpec((1, D), lambda i: (0, 0)),
            pl.BlockSpec((D, H), lambda i: (0, 0)),
            pl.BlockSpec((1, H), lambda i: (0, 0)),
            pl.BlockSpec((H, E), lambda i: (0, 0)),
            pl.BlockSpec((1, E), lambda i: (0, 0)),
        ],
        out_specs=[
            pl.BlockSpec(
                (E, 1, TC, D),
                lambda i: (0, jnp.maximum(i - P, 0) // NSB,
                           jnp.maximum(i - P, 0) % NSB, 0)),
            pl.BlockSpec((B, E), lambda i: (0, 0)),
        ],
        out_shape=[
            jax.ShapeDtypeStruct((E, B, S, D), jnp.float32),
            jax.ShapeDtypeStruct((B, E), jnp.float32),
        ],
        scratch_shapes=[
            pltpu.VMEM((B * S, D), jnp.bfloat16),
            pltpu.VMEM((PH1, D), jnp.float32),
            pltpu.VMEM((B, E), jnp.float32),
        ],
        compiler_params=pltpu.CompilerParams(
            dimension_semantics=("arbitrary",)),
    )(xf, gate_W.astype(jnp.bfloat16), gate_b.reshape(1, D),
      W1.astype(jnp.bfloat16), b1.reshape(1, H), W2, b2.reshape(1, E))

    return routed, probs


# speedup vs baseline: 1.2933x; 1.0672x over previous
import jax
import jax.numpy as jnp
from jax.experimental import pallas as pl
from jax.experimental.pallas import tpu as pltpu

D = 2048
H = 256
E = 8
B = 2
S = 2048

TM = 256
TC = 128
WQ = 4
WR = D // WQ
PH1 = B * S // TM
P = S // TM
NSB = S // TC
NOUT = B * NSB
OFF = WQ + P


def _mega_kernel(x_ref, w_ref, gb_ref, w1_ref, b1_ref, w2_ref, b2_ref,
                 routed_ref, probs_ref,
                 wb_s, gated_s, psum_s, gains_s):
    i = pl.program_id(0)

    for q in range(WQ):
        @pl.when(i == q)
        def _cast_w(q=q):
            wb_s[q * WR:(q + 1) * WR, :] = w_ref[...].astype(jnp.bfloat16)

    @pl.when(jnp.logical_and(i >= WQ, i < WQ + PH1))
    def _mm():
        k = i - WQ
        xt = x_ref[...]
        z = jnp.dot(xt.astype(jnp.bfloat16), wb_s[...],
                    preferred_element_type=jnp.float32) + gb_ref[...]
        g = xt * jax.nn.sigmoid(z)
        gated_s[pl.ds(k * TM, TM), :] = g.astype(jnp.bfloat16)
        colsum = jnp.sum(g, axis=0, keepdims=True)

        for r in range(PH1):
            if r in (P - 1, 2 * P - 1):
                continue
            @pl.when(k == r)
            def _store(r=r):
                psum_s[r:r + 1, :] = colsum

        for bb, last in ((0, P - 1), (1, 2 * P - 1)):
            @pl.when(k == last)
            def _router(bb=bb, last=last):
                ps = psum_s[...]
                prev = jnp.sum(ps[last - (P - 1):last, :], axis=0,
                               keepdims=True)
                pooled = (prev + colsum) * (1.0 / S)
                h = jnp.tanh(
                    jnp.dot(pooled.astype(jnp.bfloat16), w1_ref[...],
                            preferred_element_type=jnp.float32) + b1_ref[...])
                logits = (jnp.dot(h.astype(jnp.bfloat16),
                                  w2_ref[...].astype(jnp.bfloat16),
                                  preferred_element_type=jnp.float32)
                          + b2_ref[...])
                ids = jax.lax.broadcasted_iota(jnp.int32, (1, E), 1)
                v1 = jnp.max(logits, axis=1, keepdims=True)
                i1 = jnp.min(jnp.where(logits == v1, ids, E),
                             axis=1, keepdims=True)
                m1 = ids == i1
                masked = jnp.where(m1, -jnp.inf, logits)
                v2 = jnp.max(masked, axis=1, keepdims=True)
                i2 = jnp.min(jnp.where(masked == v2, ids, E),
                             axis=1, keepdims=True)
                m2 = ids == i2
                ex = jnp.exp(logits - v1)
                probs_ref[bb:bb + 1, :] = ex / jnp.sum(ex, axis=1,
                                                       keepdims=True)
                e2 = jnp.exp(v2 - v1)
                wt1 = 1.0 / (1.0 + e2)
                wt2 = e2 * wt1
                gains_s[bb:bb + 1, :] = (jnp.where(m1, wt1, 0.0)
                                         + jnp.where(m2, wt2, 0.0))

    @pl.when(i >= OFF)
    def _out():
        jj = i - OFF
        bb = jj // NSB
        g = gated_s[pl.ds(jj * TC, TC), :].astype(jnp.float32)
        gv = gains_s[...]
        rid = jax.lax.broadcasted_iota(jnp.int32, (B, E), 0)
        row = jnp.sum(jnp.where(rid == bb, gv, 0.0), axis=0,
                      keepdims=True)
        for e in range(E):
            routed_ref[e, 0] = g * row[0, e]


def kernel(x, gate_W, gate_b, W1, b1, W2, b2):
    xf = x.reshape(B * S, D)

    routed, probs = pl.pallas_call(
        _mega_kernel,
        grid=(OFF + NOUT,),
        in_specs=[
            pl.BlockSpec((TM, D),
                         lambda i: (jnp.clip(i - WQ, 0, PH1 - 1), 0)),
            pl.BlockSpec((WR, D), lambda i: (jnp.minimum(i, WQ - 1), 0)),
            pl.BlockSpec((1, D), lambda i: (0, 0)),
            pl.BlockSpec((D, H), lambda i: (0, 0)),
            pl.BlockSpec((1, H), lambda i: (0, 0)),
            pl.BlockSpec((H, E), lambda i: (0, 0)),
            pl.BlockSpec((1, E), lambda i: (0, 0)),
        ],
        out_specs=[
            pl.BlockSpec(
                (E, 1, TC, D),
                lambda i: (0, jnp.maximum(i - OFF, 0) // NSB,
                           jnp.maximum(i - OFF, 0) % NSB, 0)),
            pl.BlockSpec((B, E), lambda i: (0, 0)),
        ],
        out_shape=[
            jax.ShapeDtypeStruct((E, B, S, D), jnp.float32),
            jax.ShapeDtypeStruct((B, E), jnp.float32),
        ],
        scratch_shapes=[
            pltpu.VMEM((D, D), jnp.bfloat16),
            pltpu.VMEM((B * S, D), jnp.bfloat16),
            pltpu.VMEM((PH1, D), jnp.float32),
            pltpu.VMEM((B, E), jnp.float32),
        ],
        compiler_params=pltpu.CompilerParams(
            dimension_semantics=("arbitrary",)),
    )(xf, gate_W, gate_b.reshape(1, D), W1.astype(jnp.bfloat16),
      b1.reshape(1, H), W2, b2.reshape(1, E))

    return routed, probs
